# baseline (speedup 1.0000x reference)
import functools

import jax
import jax.numpy as jnp
from jax import lax
from jax.experimental import pallas as pl
from jax.experimental.pallas import tpu as pltpu
from jax.experimental.pallas import tpu_sc as plsc

EPS = 1e-5
NC = 2
NS = 16
NW = NC * NS
CHUNK = 128
NBUF = 2


def _iota_fill(buf, off):
    iota16 = lax.iota(jnp.int32, 16)
    for j in range(CHUNK // 16):
        buf[pl.ds(off + j * 16, 16)] = iota16 + j * 16


def _sc_agg_rows(n_pad, h, e):
    nchunk = e // CHUNK
    assert nchunk * CHUNK == e
    rows_per_tile = n_pad // NS
    zb = 8
    kpw = nchunk // NW
    rem = nchunk - kpw * NW
    slots = kpw + (1 if rem else 0)
    assert slots % NBUF == 0 and slots >= 2 * NBUF

    mesh = plsc.VectorSubcoreMesh(core_axis_name="c", subcore_axis_name="s")

    @functools.partial(
        pl.kernel,
        out_type=(
            jax.ShapeDtypeStruct((NC, n_pad, h), jnp.float32),
            jax.ShapeDtypeStruct((NC, n_pad), jnp.float32),
        ),
        mesh=mesh,
        scratch_types=[
            pltpu.VMEM((slots * CHUNK,), jnp.int32),
            pltpu.VMEM((slots * CHUNK,), jnp.int32),
            pltpu.VMEM((CHUNK,), jnp.float32),
            pltpu.VMEM((zb, h), jnp.float32),
            pltpu.VMEM_SHARED((n_pad, h), jnp.float32),
            pltpu.VMEM_SHARED((n_pad,), jnp.float32),
        ] + [pltpu.VMEM((CHUNK, h), jnp.float32)] * NBUF
          + [pltpu.VMEM((CHUNK,), jnp.int32)] * NBUF
          + [pltpu.SemaphoreType.DMA] * (3 * NBUF),
    )
    def k(y_hbm, ei_hbm, out_sum, out_deg,
          sbuf, dbuf, ones_v, zblk, acc_sh, deg_sh, *ring):
        rows = ring[:NBUF]
        didx = ring[NBUF:2 * NBUF]
        gsem = ring[2 * NBUF:3 * NBUF]
        ssem = ring[3 * NBUF:4 * NBUF]
        dsem = ring[4 * NBUF:]
        cid = lax.axis_index("c")
        sid = lax.axis_index("s")
        wid = sid * NC + cid
        zero16 = jnp.zeros((16,), jnp.float32)
        for r in range(zb):
            for j in range(h // 16):
                zblk[r, pl.ds(j * 16, 16)] = zero16
        for j in range(CHUNK // 16):
            ones_v[pl.ds(j * 16, 16)] = jnp.ones((16,), jnp.float32)
        base = wid * (kpw * CHUNK)
        if rem:
            _iota_fill(sbuf, kpw * CHUNK)
        pltpu.sync_copy(ei_hbm.at[0, pl.ds(base, kpw * CHUNK)],
                        sbuf.at[pl.ds(0, kpw * CHUNK)])
        pltpu.sync_copy(ei_hbm.at[1, pl.ds(base, kpw * CHUNK)],
                        dbuf.at[pl.ds(0, kpw * CHUNK)])
        if rem:
            @pl.when(wid < rem)
            def _():
                xb = (kpw * NW + wid) * CHUNK
                pltpu.sync_copy(ei_hbm.at[0, pl.ds(xb, CHUNK)],
                                sbuf.at[pl.ds(kpw * CHUNK, CHUNK)])
                pltpu.sync_copy(ei_hbm.at[1, pl.ds(xb, CHUNK)],
                                dbuf.at[pl.ds(kpw * CHUNK, CHUNK)])
        row0 = sid * rows_per_tile
        def zbody(t, c):
            pltpu.sync_copy(zblk, acc_sh.at[pl.ds(row0 + t * zb, zb)])
            return c
        lax.fori_loop(0, rows_per_tile // zb, zbody, 0)
        for t in range(rows_per_tile // CHUNK):
            pltpu.sync_copy(zblk.at[0, pl.ds(0, CHUNK)],
                            deg_sh.at[pl.ds(row0 + t * CHUNK, CHUNK)])
        plsc.subcore_barrier()

        def fill_didx(t, b):
            for j in range(CHUNK // 16):
                didx[b][pl.ds(j * 16, 16)] = dbuf[pl.ds(t * CHUNK + j * 16, 16)]

        def g_copy(t, b):
            return pltpu.make_async_copy(
                y_hbm.at[sbuf.at[pl.ds(t * CHUNK, CHUNK)]], rows[b], gsem[b])

        def s_copy(b):
            return pltpu.make_async_copy(rows[b], acc_sh.at[didx[b]], ssem[b])

        def d_copy(b):
            return pltpu.make_async_copy(ones_v, deg_sh.at[didx[b]], dsem[b])

        for b in range(NBUF):
            g_copy(b, b).start()

        def gbody(g, c):
            t0 = g * NBUF
            for b in range(NBUF):
                g_copy(t0 + b, b).wait()
                fill_didx(t0 + b, b)
                s_copy(b).start(add=True)
                d_copy(b).start(add=True)
            for b in range(NBUF):
                s_copy(b).wait()
                d_copy(b).wait()
                g_copy(t0 + NBUF + b, b).start()
            return c
        lax.fori_loop(0, slots // NBUF - 1, gbody, 0)
        tl0 = slots - NBUF
        for b in range(NBUF):
            t = tl0 + b
            g_copy(t, b).wait()
            fill_didx(t, b)
            if rem and t == slots - 1:
                @pl.when(wid < rem)
                def _(b=b):
                    s_copy(b).start(add=True)
                    d_copy(b).start(add=True)
            else:
                s_copy(b).start(add=True)
                d_copy(b).start(add=True)
        for b in range(NBUF):
            t = tl0 + b
            if rem and t == slots - 1:
                @pl.when(wid < rem)
                def _(b=b):
                    s_copy(b).wait()
                    d_copy(b).wait()
            else:
                s_copy(b).wait()
                d_copy(b).wait()
        plsc.subcore_barrier()
        pltpu.sync_copy(acc_sh.at[pl.ds(row0, rows_per_tile)],
                        out_sum.at[cid, pl.ds(row0, rows_per_tile)])
        pltpu.sync_copy(deg_sh.at[pl.ds(row0, rows_per_tile)],
                        out_deg.at[cid, pl.ds(row0, rows_per_tile)])

    return k


def _sc_agg_scalar(n_pad, e):
    nchunk = e // CHUNK
    assert nchunk * CHUNK == e
    rows_per_tile = n_pad // NS
    kpw = nchunk // NW
    rem = nchunk - kpw * NW
    slots = kpw + (1 if rem else 0)
    assert slots % NBUF == 0 and slots >= 2 * NBUF

    mesh = plsc.VectorSubcoreMesh(core_axis_name="c", subcore_axis_name="s")

    @functools.partial(
        pl.kernel,
        out_type=jax.ShapeDtypeStruct((NC, n_pad), jnp.float32),
        mesh=mesh,
        scratch_types=[
            pltpu.VMEM((slots * CHUNK,), jnp.int32),
            pltpu.VMEM((slots * CHUNK,), jnp.int32),
            pltpu.VMEM((CHUNK,), jnp.float32),
            pltpu.VMEM_SHARED((n_pad,), jnp.float32),
        ] + [pltpu.VMEM((CHUNK,), jnp.float32)] * NBUF
          + [pltpu.VMEM((CHUNK,), jnp.int32)] * NBUF
          + [pltpu.SemaphoreType.DMA] * (2 * NBUF),
    )
    def k(z_hbm, ei_hbm, out_sum, sbuf, dbuf, zrow, acc_sh, *ring):
        vals = ring[:NBUF]
        didx = ring[NBUF:2 * NBUF]
        gsem = ring[2 * NBUF:3 * NBUF]
        ssem = ring[3 * NBUF:]
        cid = lax.axis_index("c")
        sid = lax.axis_index("s")
        wid = sid * NC + cid
        zero16 = jnp.zeros((16,), jnp.float32)
        for j in range(CHUNK // 16):
            zrow[pl.ds(j * 16, 16)] = zero16
        base = wid * (kpw * CHUNK)
        if rem:
            _iota_fill(sbuf, kpw * CHUNK)
        pltpu.sync_copy(ei_hbm.at[0, pl.ds(base, kpw * CHUNK)],
                        sbuf.at[pl.ds(0, kpw * CHUNK)])
        pltpu.sync_copy(ei_hbm.at[1, pl.ds(base, kpw * CHUNK)],
                        dbuf.at[pl.ds(0, kpw * CHUNK)])
        if rem:
            @pl.when(wid < rem)
            def _():
                xb = (kpw * NW + wid) * CHUNK
                pltpu.sync_copy(ei_hbm.at[0, pl.ds(xb, CHUNK)],
                                sbuf.at[pl.ds(kpw * CHUNK, CHUNK)])
                pltpu.sync_copy(ei_hbm.at[1, pl.ds(xb, CHUNK)],
                                dbuf.at[pl.ds(kpw * CHUNK, CHUNK)])
        row0 = sid * rows_per_tile
        for t in range(rows_per_tile // CHUNK):
            pltpu.sync_copy(zrow, acc_sh.at[pl.ds(row0 + t * CHUNK, CHUNK)])
        plsc.subcore_barrier()

        def fill_didx(t, b):
            for j in range(CHUNK // 16):
                didx[b][pl.ds(j * 16, 16)] = dbuf[pl.ds(t * CHUNK + j * 16, 16)]

        def g_copy(t, b):
            return pltpu.make_async_copy(
                z_hbm.at[sbuf.at[pl.ds(t * CHUNK, CHUNK)]], vals[b], gsem[b])

        def s_copy(b):
            return pltpu.make_async_copy(vals[b], acc_sh.at[didx[b]], ssem[b])

        for b in range(NBUF):
            g_copy(b, b).start()

        def gbody(g, c):
            t0 = g * NBUF
            for b in range(NBUF):
                g_copy(t0 + b, b).wait()
                fill_didx(t0 + b, b)
                s_copy(b).start(add=True)
            for b in range(NBUF):
                s_copy(b).wait()
                g_copy(t0 + NBUF + b, b).start()
            return c
        lax.fori_loop(0, slots // NBUF - 1, gbody, 0)
        tl0 = slots - NBUF
        for b in range(NBUF):
            t = tl0 + b
            g_copy(t, b).wait()
            fill_didx(t, b)
            if rem and t == slots - 1:
                @pl.when(wid < rem)
                def _(b=b):
                    s_copy(b).start(add=True)
            else:
                s_copy(b).start(add=True)
        for b in range(NBUF):
            t = tl0 + b
            if rem and t == slots - 1:
                @pl.when(wid < rem)
                def _(b=b):
                    s_copy(b).wait()
            else:
                s_copy(b).wait()
        plsc.subcore_barrier()
        pltpu.sync_copy(acc_sh.at[pl.ds(row0, rows_per_tile)],
                        out_sum.at[cid, pl.ds(row0, rows_per_tile)])

    return k


def _mm_body(x_ref, w_ref, y_ref):
    dn = (((1,), (1,)), ((), ()))
    y_ref[...] = lax.dot_general(x_ref[...], w_ref[...], dn,
                                 preferred_element_type=jnp.float32)


def _k2_body(ps0_ref, ps1_ref, yr_ref, pd0_ref, pd1_ref, b1_ref,
             bnw_ref, bnb_ref, bnm_ref, bnv_ref,
             w2l_ref, w2r_ref, wp_ref, b2_ref, bp_ref,
             zl_ref, zr_ref):
    deg = jnp.maximum(pd0_ref[0] + pd1_ref[0], 1.0)
    agg = (ps0_ref[0] + ps1_ref[0]) / deg
    c = agg + b1_ref[...] + yr_ref[...]
    scale = bnw_ref[...] * lax.rsqrt(bnv_ref[...] + EPS)
    h1 = jnp.maximum(scale * (c - bnm_ref[...]) + bnb_ref[...], 0.0)
    dn_c0 = (((0,), (1,)), ((), ()))
    v_l = lax.dot_general(w2l_ref[...], wp_ref[...], dn_c0,
                          preferred_element_type=jnp.float32)
    v_r = lax.dot_general(w2r_ref[...], wp_ref[...], dn_c0,
                          preferred_element_type=jnp.float32)
    dn_r = (((1,), (0,)), ((), ()))
    zl_ref[...] = lax.dot_general(h1, v_l, dn_r,
                                  preferred_element_type=jnp.float32)
    cst = jnp.sum(b2_ref[...] * wp_ref[...]) + bp_ref[0, 0]
    zr_ref[...] = lax.dot_general(h1, v_r, dn_r,
                                  preferred_element_type=jnp.float32) + cst


def _k3_body(p0_ref, p1_ref, pd0_ref, pd1_ref, zr_ref, out_ref):
    deg = jnp.maximum(pd0_ref[0] + pd1_ref[0], 1.0)
    out_ref[...] = (p0_ref[0] + p1_ref[0]) / deg + zr_ref[...]


def kernel(node_feature, edge_index, batch, W1_l, b1, W1_r,
           bn1_w, bn1_b, bn1_m, bn1_v, W2_l, b2, W2_r, Wp, bp):
    n, d_in = node_feature.shape
    h = W1_l.shape[0]
    e = edge_index.shape[1]
    n_pad = ((n + NS * CHUNK - 1) // (NS * CHUNK)) * (NS * CHUNK)

    f32 = jnp.float32
    bm1, bm2, bm3 = 1000, 1000, 2000

    def _mm(w):
        return pl.pallas_call(
            _mm_body,
            grid=(n // bm1,),
            in_specs=[
                pl.BlockSpec((bm1, d_in), lambda i: (i, 0)),
                pl.BlockSpec((h, d_in), lambda i: (0, 0)),
            ],
            out_specs=pl.BlockSpec((bm1, h), lambda i: (i, 0)),
            out_shape=jax.ShapeDtypeStruct((n, h), f32),
        )(node_feature, w)
    y_l = _mm(W1_l)
    y_r = _mm(W1_r)

    psum, pdeg = _sc_agg_rows(n_pad, h, e)(y_l, edge_index)
    pdeg3 = pdeg.reshape(NC, n_pad, 1)

    full = lambda r, c: pl.BlockSpec((r, c), lambda i: (0, 0))
    zl, zr = pl.pallas_call(
        _k2_body,
        grid=(n // bm2,),
        in_specs=[
            pl.BlockSpec((1, bm2, h), lambda i: (0, i, 0)),
            pl.BlockSpec((1, bm2, h), lambda i: (1, i, 0)),
            pl.BlockSpec((bm2, h), lambda i: (i, 0)),
            pl.BlockSpec((1, bm2, 1), lambda i: (0, i, 0)),
            pl.BlockSpec((1, bm2, 1), lambda i: (1, i, 0)),
            full(1, h), full(1, h), full(1, h), full(1, h), full(1, h),
            full(h, h), full(h, h), full(1, h), full(1, h), full(1, 1),
        ],
        out_specs=[
            pl.BlockSpec((bm2, 1), lambda i: (i, 0)),
            pl.BlockSpec((bm2, 1), lambda i: (i, 0)),
        ],
        out_shape=[
            jax.ShapeDtypeStruct((n, 1), f32),
            jax.ShapeDtypeStruct((n, 1), f32),
        ],
    )(psum, psum, y_r, pdeg3, pdeg3,
      b1.reshape(1, h), bn1_w.reshape(1, h), bn1_b.reshape(1, h),
      bn1_m.reshape(1, h), bn1_v.reshape(1, h),
      W2_l, W2_r, Wp, b2.reshape(1, h), bp.reshape(1, 1))

    p2 = _sc_agg_scalar(n_pad, e)(zl.reshape(n), edge_index)
    p23 = p2.reshape(NC, n_pad, 1)

    out = pl.pallas_call(
        _k3_body,
        grid=(n // bm3,),
        in_specs=[
            pl.BlockSpec((1, bm3, 1), lambda i: (0, i, 0)),
            pl.BlockSpec((1, bm3, 1), lambda i: (1, i, 0)),
            pl.BlockSpec((1, bm3, 1), lambda i: (0, i, 0)),
            pl.BlockSpec((1, bm3, 1), lambda i: (1, i, 0)),
            pl.BlockSpec((bm3, 1), lambda i: (i, 0)),
        ],
        out_specs=pl.BlockSpec((bm3, 1), lambda i: (i, 0)),
        out_shape=jax.ShapeDtypeStruct((n, 1), f32),
    )(p23, p23, pdeg3, pdeg3, zr)
    return out


# speedup vs baseline: 1.0140x; 1.0140x over previous
import functools

import jax
import jax.numpy as jnp
from jax import lax
from jax.experimental import pallas as pl
from jax.experimental.pallas import tpu as pltpu
from jax.experimental.pallas import tpu_sc as plsc

EPS = 1e-5
NC = 2
NS = 16
NW = NC * NS
CHUNK = 128
BM = 400


NBUF = 2


def _sc_agg_rows(n_pad, h, e):
    nchunk = e // CHUNK
    rows_per_tile = n_pad // NS
    zb = 8
    kpw = nchunk // NW
    assert kpw * NW == nchunk and kpw % 8 == 0
    slots = kpw
    ngroups = slots // NBUF

    mesh = plsc.VectorSubcoreMesh(core_axis_name="c", subcore_axis_name="s")

    @functools.partial(
        pl.kernel,
        out_type=(
            jax.ShapeDtypeStruct((NC, n_pad, h), jnp.float32),
            jax.ShapeDtypeStruct((NC, n_pad), jnp.float32),
        ),
        mesh=mesh,
        scratch_types=[
            pltpu.VMEM((slots, CHUNK), jnp.int32),
            pltpu.VMEM((slots, CHUNK), jnp.int32),
            pltpu.VMEM((CHUNK,), jnp.float32),
            pltpu.VMEM((zb, h), jnp.float32),
            pltpu.VMEM_SHARED((n_pad, h), jnp.float32),
            pltpu.VMEM_SHARED((n_pad,), jnp.float32),
        ] + [pltpu.VMEM((CHUNK, h), jnp.float32)] * NBUF
          + [pltpu.SemaphoreType.DMA] * (3 * NBUF),
    )
    def k(y_hbm, src_hbm, dst_hbm, out_sum, out_deg,
          sbuf, dbuf, ones_v, zblk, acc_sh, deg_sh, *ring):
        rows = ring[:NBUF]
        gsem = ring[NBUF:2 * NBUF]
        ssem = ring[2 * NBUF:3 * NBUF]
        dsem = ring[3 * NBUF:]
        cid = lax.axis_index("c")
        sid = lax.axis_index("s")
        wid = sid * NC + cid
        zero16 = jnp.zeros((16,), jnp.float32)
        one16 = jnp.ones((16,), jnp.float32)
        for r in range(zb):
            for j in range(h // 16):
                zblk[r, pl.ds(j * 16, 16)] = zero16
        for j in range(CHUNK // 16):
            ones_v[pl.ds(j * 16, 16)] = one16
        pltpu.sync_copy(src_hbm.at[pl.ds(wid * kpw, kpw)],
                        sbuf.at[pl.ds(0, kpw)])
        pltpu.sync_copy(dst_hbm.at[pl.ds(wid * kpw, kpw)],
                        dbuf.at[pl.ds(0, kpw)])
        row0 = sid * rows_per_tile
        def zbody(t, c):
            pltpu.sync_copy(zblk, acc_sh.at[pl.ds(row0 + t * zb, zb)])
            return c
        lax.fori_loop(0, rows_per_tile // zb, zbody, 0)
        for t in range(rows_per_tile // CHUNK):
            pltpu.sync_copy(zblk.at[0, pl.ds(0, CHUNK)],
                            deg_sh.at[pl.ds(row0 + t * CHUNK, CHUNK)])
        plsc.subcore_barrier()

        for b in range(NBUF):
            pltpu.make_async_copy(
                y_hbm.at[sbuf.at[b]], rows[b], gsem[b]).start()

        def gbody(g, c):
            t0 = g * NBUF
            for b in range(NBUF):
                pltpu.make_async_copy(
                    y_hbm.at[sbuf.at[t0 + b]], rows[b], gsem[b]).wait()
                pltpu.make_async_copy(
                    rows[b], acc_sh.at[dbuf.at[t0 + b]], ssem[b]).start(add=True)
                pltpu.make_async_copy(
                    ones_v, deg_sh.at[dbuf.at[t0 + b]], dsem[b]).start(add=True)
            for b in range(NBUF):
                nt = t0 + NBUF + b
                pltpu.make_async_copy(
                    rows[b], acc_sh.at[dbuf.at[t0 + b]], ssem[b]).wait()
                pltpu.make_async_copy(
                    ones_v, deg_sh.at[dbuf.at[t0 + b]], dsem[b]).wait()
                pltpu.make_async_copy(
                    y_hbm.at[sbuf.at[nt]], rows[b], gsem[b]).start()
            return c
        lax.fori_loop(0, ngroups - 1, gbody, 0)
        tl0 = (ngroups - 1) * NBUF
        for b in range(NBUF):
            pltpu.make_async_copy(
                y_hbm.at[sbuf.at[tl0 + b]], rows[b], gsem[b]).wait()
            pltpu.make_async_copy(
                rows[b], acc_sh.at[dbuf.at[tl0 + b]], ssem[b]).start(add=True)
            pltpu.make_async_copy(
                ones_v, deg_sh.at[dbuf.at[tl0 + b]], dsem[b]).start(add=True)
        for b in range(NBUF):
            pltpu.make_async_copy(
                rows[b], acc_sh.at[dbuf.at[tl0 + b]], ssem[b]).wait()
            pltpu.make_async_copy(
                ones_v, deg_sh.at[dbuf.at[tl0 + b]], dsem[b]).wait()
        plsc.subcore_barrier()
        pltpu.sync_copy(acc_sh.at[pl.ds(row0, rows_per_tile)],
                        out_sum.at[cid, pl.ds(row0, rows_per_tile)])
        pltpu.sync_copy(deg_sh.at[pl.ds(row0, rows_per_tile)],
                        out_deg.at[cid, pl.ds(row0, rows_per_tile)])

    return k


def _sc_agg_scalar(n_pad, e):
    nchunk = e // CHUNK
    rows_per_tile = n_pad // NS
    kpw = nchunk // NW
    assert kpw * NW == nchunk and kpw % 8 == 0
    slots = kpw
    ngroups = slots // NBUF

    mesh = plsc.VectorSubcoreMesh(core_axis_name="c", subcore_axis_name="s")

    @functools.partial(
        pl.kernel,
        out_type=jax.ShapeDtypeStruct((NC, n_pad), jnp.float32),
        mesh=mesh,
        scratch_types=[
            pltpu.VMEM((slots, CHUNK), jnp.int32),
            pltpu.VMEM((slots, CHUNK), jnp.int32),
            pltpu.VMEM((CHUNK,), jnp.float32),
            pltpu.VMEM_SHARED((n_pad,), jnp.float32),
        ] + [pltpu.VMEM((CHUNK,), jnp.float32)] * NBUF
          + [pltpu.SemaphoreType.DMA] * (2 * NBUF),
    )
    def k(z_hbm, src_hbm, dst_hbm, out_sum, sbuf, dbuf, zrow, acc_sh, *ring):
        vals = ring[:NBUF]
        gsem = ring[NBUF:2 * NBUF]
        ssem = ring[2 * NBUF:]
        cid = lax.axis_index("c")
        sid = lax.axis_index("s")
        wid = sid * NC + cid
        zero16 = jnp.zeros((16,), jnp.float32)
        for j in range(CHUNK // 16):
            zrow[pl.ds(j * 16, 16)] = zero16
        pltpu.sync_copy(src_hbm.at[pl.ds(wid * kpw, kpw)],
                        sbuf.at[pl.ds(0, kpw)])
        pltpu.sync_copy(dst_hbm.at[pl.ds(wid * kpw, kpw)],
                        dbuf.at[pl.ds(0, kpw)])
        row0 = sid * rows_per_tile
        for t in range(rows_per_tile // CHUNK):
            pltpu.sync_copy(zrow, acc_sh.at[pl.ds(row0 + t * CHUNK, CHUNK)])
        plsc.subcore_barrier()

        for b in range(NBUF):
            pltpu.make_async_copy(
                z_hbm.at[sbuf.at[b]], vals[b], gsem[b]).start()

        def gbody(g, c):
            t0 = g * NBUF
            for b in range(NBUF):
                pltpu.make_async_copy(
                    z_hbm.at[sbuf.at[t0 + b]], vals[b], gsem[b]).wait()
                pltpu.make_async_copy(
                    vals[b], acc_sh.at[dbuf.at[t0 + b]], ssem[b]).start(add=True)
            for b in range(NBUF):
                pltpu.make_async_copy(
                    vals[b], acc_sh.at[dbuf.at[t0 + b]], ssem[b]).wait()
                pltpu.make_async_copy(
                    z_hbm.at[sbuf.at[t0 + NBUF + b]], vals[b], gsem[b]).start()
            return c
        lax.fori_loop(0, ngroups - 1, gbody, 0)
        tl0 = (ngroups - 1) * NBUF
        for b in range(NBUF):
            pltpu.make_async_copy(
                z_hbm.at[sbuf.at[tl0 + b]], vals[b], gsem[b]).wait()
            pltpu.make_async_copy(
                vals[b], acc_sh.at[dbuf.at[tl0 + b]], ssem[b]).start(add=True)
        for b in range(NBUF):
            pltpu.make_async_copy(
                vals[b], acc_sh.at[dbuf.at[tl0 + b]], ssem[b]).wait()
        plsc.subcore_barrier()
        pltpu.sync_copy(acc_sh.at[pl.ds(row0, rows_per_tile)],
                        out_sum.at[cid, pl.ds(row0, rows_per_tile)])

    return k


def _mm_body(x_ref, w_ref, y_ref):
    dn = (((1,), (1,)), ((), ()))
    y_ref[...] = lax.dot_general(x_ref[...], w_ref[...], dn,
                                 preferred_element_type=jnp.float32)


def _k2_body(ps0_ref, ps1_ref, yr_ref, pd0_ref, pd1_ref, b1_ref,
             bnw_ref, bnb_ref, bnm_ref, bnv_ref,
             w2l_ref, w2r_ref, wp_ref, b2_ref, bp_ref,
             zl_ref, zr_ref):
    deg = jnp.maximum(pd0_ref[0] + pd1_ref[0], 1.0)
    agg = (ps0_ref[0] + ps1_ref[0]) / deg
    c = agg + b1_ref[...] + yr_ref[...]
    scale = bnw_ref[...] * lax.rsqrt(bnv_ref[...] + EPS)
    h1 = jnp.maximum(scale * (c - bnm_ref[...]) + bnb_ref[...], 0.0)
    dn_c0 = (((0,), (1,)), ((), ()))
    v_l = lax.dot_general(w2l_ref[...], wp_ref[...], dn_c0,
                          preferred_element_type=jnp.float32)
    v_r = lax.dot_general(w2r_ref[...], wp_ref[...], dn_c0,
                          preferred_element_type=jnp.float32)
    dn_r = (((1,), (0,)), ((), ()))
    zl_ref[...] = lax.dot_general(h1, v_l, dn_r,
                                  preferred_element_type=jnp.float32)
    cst = jnp.sum(b2_ref[...] * wp_ref[...]) + bp_ref[0, 0]
    zr_ref[...] = lax.dot_general(h1, v_r, dn_r,
                                  preferred_element_type=jnp.float32) + cst


def _k3_body(p0_ref, p1_ref, pd0_ref, pd1_ref, zr_ref, out_ref):
    deg = jnp.maximum(pd0_ref[0] + pd1_ref[0], 1.0)
    out_ref[...] = (p0_ref[0] + p1_ref[0]) / deg + zr_ref[...]


def kernel(node_feature, edge_index, batch, W1_l, b1, W1_r,
           bn1_w, bn1_b, bn1_m, bn1_v, W2_l, b2, W2_r, Wp, bp):
    n, d_in = node_feature.shape
    h = W1_l.shape[0]
    e = edge_index.shape[1]
    n_pad = ((n + NS * CHUNK - 1) // (NS * CHUNK)) * (NS * CHUNK)
    if n_pad == n:
        n_pad += NS * CHUNK
    src = edge_index[0]
    dst = edge_index[1]

    f32 = jnp.float32
    bm1, bm2, bm3 = 1000, 1000, 2000

    def _mm(w):
        return pl.pallas_call(
            _mm_body,
            grid=(n // bm1,),
            in_specs=[
                pl.BlockSpec((bm1, d_in), lambda i: (i, 0)),
                pl.BlockSpec((h, d_in), lambda i: (0, 0)),
            ],
            out_specs=pl.BlockSpec((bm1, h), lambda i: (i, 0)),
            out_shape=jax.ShapeDtypeStruct((n, h), f32),
        )(node_feature, w)
    y_l = _mm(W1_l)
    y_r = _mm(W1_r)

    nchunk = e // CHUNK
    nchunk_pad = -(-e // (CHUNK * NW * 8)) * (NW * 8)
    e_pad = nchunk_pad * CHUNK

    def _pad_body(s_ref, d_ref, so_ref, do_ref):
        pr = nchunk_pad - nchunk
        ii = (lax.broadcasted_iota(jnp.int32, (pr, CHUNK), 0) * CHUNK
              + lax.broadcasted_iota(jnp.int32, (pr, CHUNK), 1))
        so_ref[...] = jnp.concatenate([s_ref[...], lax.rem(ii, n)])
        do_ref[...] = jnp.concatenate(
            [d_ref[...], n + lax.rem(ii, n_pad - n)])

    src2d, dst2d = pl.pallas_call(
        _pad_body,
        out_shape=[jax.ShapeDtypeStruct((nchunk_pad, CHUNK), jnp.int32)] * 2,
    )(src.reshape(nchunk, CHUNK), dst.reshape(nchunk, CHUNK))

    psum, pdeg = _sc_agg_rows(n_pad, h, e_pad)(y_l, src2d, dst2d)
    pdeg3 = pdeg.reshape(NC, n_pad, 1)

    full = lambda r, c: pl.BlockSpec((r, c), lambda i: (0, 0))
    zl, zr = pl.pallas_call(
        _k2_body,
        grid=(n // bm2,),
        in_specs=[
            pl.BlockSpec((1, bm2, h), lambda i: (0, i, 0)),
            pl.BlockSpec((1, bm2, h), lambda i: (1, i, 0)),
            pl.BlockSpec((bm2, h), lambda i: (i, 0)),
            pl.BlockSpec((1, bm2, 1), lambda i: (0, i, 0)),
            pl.BlockSpec((1, bm2, 1), lambda i: (1, i, 0)),
            full(1, h), full(1, h), full(1, h), full(1, h), full(1, h),
            full(h, h), full(h, h), full(1, h), full(1, h), full(1, 1),
        ],
        out_specs=[
            pl.BlockSpec((bm2, 1), lambda i: (i, 0)),
            pl.BlockSpec((bm2, 1), lambda i: (i, 0)),
        ],
        out_shape=[
            jax.ShapeDtypeStruct((n, 1), f32),
            jax.ShapeDtypeStruct((n, 1), f32),
        ],
    )(psum, psum, y_r, pdeg3, pdeg3,
      b1.reshape(1, h), bn1_w.reshape(1, h), bn1_b.reshape(1, h),
      bn1_m.reshape(1, h), bn1_v.reshape(1, h),
      W2_l, W2_r, Wp, b2.reshape(1, h), bp.reshape(1, 1))

    p2 = _sc_agg_scalar(n_pad, e_pad)(zl.reshape(n), src2d, dst2d)
    p23 = p2.reshape(NC, n_pad, 1)

    out = pl.pallas_call(
        _k3_body,
        grid=(n // bm3,),
        in_specs=[
            pl.BlockSpec((1, bm3, 1), lambda i: (0, i, 0)),
            pl.BlockSpec((1, bm3, 1), lambda i: (1, i, 0)),
            pl.BlockSpec((1, bm3, 1), lambda i: (0, i, 0)),
            pl.BlockSpec((1, bm3, 1), lambda i: (1, i, 0)),
            pl.BlockSpec((bm3, 1), lambda i: (i, 0)),
        ],
        out_specs=pl.BlockSpec((bm3, 1), lambda i: (i, 0)),
        out_shape=jax.ShapeDtypeStruct((n, 1), f32),
    )(p23, p23, pdeg3, pdeg3, zr)
    return out


# speedup vs baseline: 1.0227x; 1.0086x over previous
import functools

import jax
import jax.numpy as jnp
from jax import lax
from jax.experimental import pallas as pl
from jax.experimental.pallas import tpu as pltpu
from jax.experimental.pallas import tpu_sc as plsc

EPS = 1e-5
NC = 2
NS = 16
NW = NC * NS
CHUNK = 128
BM = 400


NBUF = 2


def _sc_agg_rows(n_pad, h, e):
    nchunk = e // CHUNK
    rows_per_tile = n_pad // NS
    zb = 8
    kpw = nchunk // NW
    assert kpw * NW == nchunk and kpw % 8 == 0
    slots = kpw
    ngroups = slots // NBUF

    mesh = plsc.VectorSubcoreMesh(core_axis_name="c", subcore_axis_name="s")

    @functools.partial(
        pl.kernel,
        out_type=(
            jax.ShapeDtypeStruct((NC, n_pad, h), jnp.float32),
            jax.ShapeDtypeStruct((NC, n_pad), jnp.float32),
        ),
        mesh=mesh,
        scratch_types=[
            pltpu.VMEM((slots, CHUNK), jnp.int32),
            pltpu.VMEM((slots, CHUNK), jnp.int32),
            pltpu.VMEM((CHUNK,), jnp.float32),
            pltpu.VMEM((zb, h), jnp.float32),
            pltpu.VMEM_SHARED((n_pad, h), jnp.float32),
            pltpu.VMEM_SHARED((n_pad,), jnp.float32),
        ] + [pltpu.VMEM((CHUNK, h), jnp.float32)] * NBUF
          + [pltpu.SemaphoreType.DMA] * (3 * NBUF),
    )
    def k(y_hbm, src_hbm, dst_hbm, out_sum, out_deg,
          sbuf, dbuf, ones_v, zblk, acc_sh, deg_sh, *ring):
        rows = ring[:NBUF]
        gsem = ring[NBUF:2 * NBUF]
        ssem = ring[2 * NBUF:3 * NBUF]
        dsem = ring[3 * NBUF:]
        cid = lax.axis_index("c")
        sid = lax.axis_index("s")
        wid = sid * NC + cid
        zero16 = jnp.zeros((16,), jnp.float32)
        one16 = jnp.ones((16,), jnp.float32)
        for r in range(zb):
            for j in range(h // 16):
                zblk[r, pl.ds(j * 16, 16)] = zero16
        for j in range(CHUNK // 16):
            ones_v[pl.ds(j * 16, 16)] = one16
        pltpu.sync_copy(src_hbm.at[pl.ds(wid * kpw, kpw)],
                        sbuf.at[pl.ds(0, kpw)])
        pltpu.sync_copy(dst_hbm.at[pl.ds(wid * kpw, kpw)],
                        dbuf.at[pl.ds(0, kpw)])
        row0 = sid * rows_per_tile
        def zbody(t, c):
            pltpu.sync_copy(zblk, acc_sh.at[pl.ds(row0 + t * zb, zb)])
            return c
        lax.fori_loop(0, rows_per_tile // zb, zbody, 0)
        for t in range(rows_per_tile // CHUNK):
            pltpu.sync_copy(zblk.at[0, pl.ds(0, CHUNK)],
                            deg_sh.at[pl.ds(row0 + t * CHUNK, CHUNK)])
        plsc.subcore_barrier()

        for b in range(NBUF):
            pltpu.make_async_copy(
                y_hbm.at[sbuf.at[b]], rows[b], gsem[b]).start()

        def gbody(g, c):
            t0 = g * NBUF
            for b in range(NBUF):
                pltpu.make_async_copy(
                    y_hbm.at[sbuf.at[t0 + b]], rows[b], gsem[b]).wait()
                pltpu.make_async_copy(
                    rows[b], acc_sh.at[dbuf.at[t0 + b]], ssem[b]).start(add=True)
                pltpu.make_async_copy(
                    ones_v, deg_sh.at[dbuf.at[t0 + b]], dsem[b]).start(add=True)
            for b in range(NBUF):
                nt = t0 + NBUF + b
                pltpu.make_async_copy(
                    rows[b], acc_sh.at[dbuf.at[t0 + b]], ssem[b]).wait()
                pltpu.make_async_copy(
                    ones_v, deg_sh.at[dbuf.at[t0 + b]], dsem[b]).wait()
                pltpu.make_async_copy(
                    y_hbm.at[sbuf.at[nt]], rows[b], gsem[b]).start()
            return c
        lax.fori_loop(0, ngroups - 1, gbody, 0)
        tl0 = (ngroups - 1) * NBUF
        for b in range(NBUF):
            pltpu.make_async_copy(
                y_hbm.at[sbuf.at[tl0 + b]], rows[b], gsem[b]).wait()
            pltpu.make_async_copy(
                rows[b], acc_sh.at[dbuf.at[tl0 + b]], ssem[b]).start(add=True)
            pltpu.make_async_copy(
                ones_v, deg_sh.at[dbuf.at[tl0 + b]], dsem[b]).start(add=True)
        for b in range(NBUF):
            pltpu.make_async_copy(
                rows[b], acc_sh.at[dbuf.at[tl0 + b]], ssem[b]).wait()
            pltpu.make_async_copy(
                ones_v, deg_sh.at[dbuf.at[tl0 + b]], dsem[b]).wait()
        plsc.subcore_barrier()
        pltpu.sync_copy(acc_sh.at[pl.ds(row0, rows_per_tile)],
                        out_sum.at[cid, pl.ds(row0, rows_per_tile)])
        pltpu.sync_copy(deg_sh.at[pl.ds(row0, rows_per_tile)],
                        out_deg.at[cid, pl.ds(row0, rows_per_tile)])

    return k


def _sc_agg_scalar(n_pad, e):
    nchunk = e // CHUNK
    rows_per_tile = n_pad // NS
    kpw = nchunk // NW
    assert kpw * NW == nchunk and kpw % 8 == 0
    slots = kpw
    ngroups = slots // NBUF

    mesh = plsc.VectorSubcoreMesh(core_axis_name="c", subcore_axis_name="s")

    @functools.partial(
        pl.kernel,
        out_type=jax.ShapeDtypeStruct((NC, n_pad), jnp.float32),
        mesh=mesh,
        scratch_types=[
            pltpu.VMEM((slots, CHUNK), jnp.int32),
            pltpu.VMEM((slots, CHUNK), jnp.int32),
            pltpu.VMEM((CHUNK,), jnp.float32),
            pltpu.VMEM_SHARED((n_pad,), jnp.float32),
        ] + [pltpu.VMEM((CHUNK,), jnp.float32)] * NBUF
          + [pltpu.SemaphoreType.DMA] * (2 * NBUF),
    )
    def k(z_hbm, src_hbm, dst_hbm, out_sum, sbuf, dbuf, zrow, acc_sh, *ring):
        vals = ring[:NBUF]
        gsem = ring[NBUF:2 * NBUF]
        ssem = ring[2 * NBUF:]
        cid = lax.axis_index("c")
        sid = lax.axis_index("s")
        wid = sid * NC + cid
        zero16 = jnp.zeros((16,), jnp.float32)
        for j in range(CHUNK // 16):
            zrow[pl.ds(j * 16, 16)] = zero16
        pltpu.sync_copy(src_hbm.at[pl.ds(wid * kpw, kpw)],
                        sbuf.at[pl.ds(0, kpw)])
        pltpu.sync_copy(dst_hbm.at[pl.ds(wid * kpw, kpw)],
                        dbuf.at[pl.ds(0, kpw)])
        row0 = sid * rows_per_tile
        for t in range(rows_per_tile // CHUNK):
            pltpu.sync_copy(zrow, acc_sh.at[pl.ds(row0 + t * CHUNK, CHUNK)])
        plsc.subcore_barrier()

        for b in range(NBUF):
            pltpu.make_async_copy(
                z_hbm.at[sbuf.at[b]], vals[b], gsem[b]).start()

        def gbody(g, c):
            t0 = g * NBUF
            for b in range(NBUF):
                pltpu.make_async_copy(
                    z_hbm.at[sbuf.at[t0 + b]], vals[b], gsem[b]).wait()
                pltpu.make_async_copy(
                    vals[b], acc_sh.at[dbuf.at[t0 + b]], ssem[b]).start(add=True)
            for b in range(NBUF):
                pltpu.make_async_copy(
                    vals[b], acc_sh.at[dbuf.at[t0 + b]], ssem[b]).wait()
                pltpu.make_async_copy(
                    z_hbm.at[sbuf.at[t0 + NBUF + b]], vals[b], gsem[b]).start()
            return c
        lax.fori_loop(0, ngroups - 1, gbody, 0)
        tl0 = (ngroups - 1) * NBUF
        for b in range(NBUF):
            pltpu.make_async_copy(
                z_hbm.at[sbuf.at[tl0 + b]], vals[b], gsem[b]).wait()
            pltpu.make_async_copy(
                vals[b], acc_sh.at[dbuf.at[tl0 + b]], ssem[b]).start(add=True)
        for b in range(NBUF):
            pltpu.make_async_copy(
                vals[b], acc_sh.at[dbuf.at[tl0 + b]], ssem[b]).wait()
        plsc.subcore_barrier()
        pltpu.sync_copy(acc_sh.at[pl.ds(row0, rows_per_tile)],
                        out_sum.at[cid, pl.ds(row0, rows_per_tile)])

    return k


def _mm_body(x_ref, w_ref, y_ref):
    dn = (((1,), (1,)), ((), ()))
    y_ref[...] = lax.dot_general(x_ref[...], w_ref[...], dn,
                                 preferred_element_type=jnp.float32)


def _k2_body(ps0_ref, ps1_ref, yr_ref, pd0_ref, pd1_ref, b1_ref,
             bnw_ref, bnb_ref, bnm_ref, bnv_ref,
             w2l_ref, w2r_ref, wp_ref, b2_ref, bp_ref,
             zl_ref, zr_ref):
    deg = jnp.maximum(pd0_ref[0] + pd1_ref[0], 1.0)
    agg = (ps0_ref[0] + ps1_ref[0]) / deg
    c = agg + b1_ref[...] + yr_ref[...]
    scale = bnw_ref[...] * lax.rsqrt(bnv_ref[...] + EPS)
    h1 = jnp.maximum(scale * (c - bnm_ref[...]) + bnb_ref[...], 0.0)
    dn_c0 = (((0,), (1,)), ((), ()))
    v_l = lax.dot_general(w2l_ref[...], wp_ref[...], dn_c0,
                          preferred_element_type=jnp.float32)
    v_r = lax.dot_general(w2r_ref[...], wp_ref[...], dn_c0,
                          preferred_element_type=jnp.float32)
    dn_r = (((1,), (0,)), ((), ()))
    zl_ref[...] = lax.dot_general(h1, v_l, dn_r,
                                  preferred_element_type=jnp.float32)
    cst = jnp.sum(b2_ref[...] * wp_ref[...]) + bp_ref[0, 0]
    zr_ref[...] = lax.dot_general(h1, v_r, dn_r,
                                  preferred_element_type=jnp.float32) + cst


def _k3_body(p0_ref, p1_ref, pd0_ref, pd1_ref, zr_ref, out_ref):
    deg = jnp.maximum(pd0_ref[0] + pd1_ref[0], 1.0)
    out_ref[...] = (p0_ref[0] + p1_ref[0]) / deg + zr_ref[...]


def kernel(node_feature, edge_index, batch, W1_l, b1, W1_r,
           bn1_w, bn1_b, bn1_m, bn1_v, W2_l, b2, W2_r, Wp, bp):
    n, d_in = node_feature.shape
    h = W1_l.shape[0]
    e = edge_index.shape[1]
    n_pad = ((n + NS * CHUNK - 1) // (NS * CHUNK)) * (NS * CHUNK)
    if n_pad == n:
        n_pad += NS * CHUNK
    src = edge_index[0]
    dst = edge_index[1]

    f32 = jnp.float32
    bm1, bm2, bm3 = 1000, 1000, 2000

    def _mm(w):
        return pl.pallas_call(
            _mm_body,
            grid=(n // bm1,),
            in_specs=[
                pl.BlockSpec((bm1, d_in), lambda i: (i, 0)),
                pl.BlockSpec((h, d_in), lambda i: (0, 0)),
            ],
            out_specs=pl.BlockSpec((bm1, h), lambda i: (i, 0)),
            out_shape=jax.ShapeDtypeStruct((n, h), f32),
        )(node_feature, w)
    y_l = _mm(W1_l)
    y_r = _mm(W1_r)

    nchunk_pad = -(-e // (CHUNK * NW * 8)) * (NW * 8)
    e_pad = nchunk_pad * CHUNK
    pad_iota = jnp.arange(e_pad - e, dtype=jnp.int32)
    pad_src = jax.lax.rem(pad_iota, jnp.int32(n))
    pad_dst = n + jax.lax.rem(pad_iota, jnp.int32(n_pad - n))
    src2d = jnp.concatenate([src, pad_src]).reshape(nchunk_pad, CHUNK)
    dst2d = jnp.concatenate([dst, pad_dst]).reshape(nchunk_pad, CHUNK)

    psum, pdeg = _sc_agg_rows(n_pad, h, e_pad)(y_l, src2d, dst2d)
    pdeg3 = pdeg.reshape(NC, n_pad, 1)

    full = lambda r, c: pl.BlockSpec((r, c), lambda i: (0, 0))
    zl, zr = pl.pallas_call(
        _k2_body,
        grid=(n // bm2,),
        in_specs=[
            pl.BlockSpec((1, bm2, h), lambda i: (0, i, 0)),
            pl.BlockSpec((1, bm2, h), lambda i: (1, i, 0)),
            pl.BlockSpec((bm2, h), lambda i: (i, 0)),
            pl.BlockSpec((1, bm2, 1), lambda i: (0, i, 0)),
            pl.BlockSpec((1, bm2, 1), lambda i: (1, i, 0)),
            full(1, h), full(1, h), full(1, h), full(1, h), full(1, h),
            full(h, h), full(h, h), full(1, h), full(1, h), full(1, 1),
        ],
        out_specs=[
            pl.BlockSpec((bm2, 1), lambda i: (i, 0)),
            pl.BlockSpec((bm2, 1), lambda i: (i, 0)),
        ],
        out_shape=[
            jax.ShapeDtypeStruct((n, 1), f32),
            jax.ShapeDtypeStruct((n, 1), f32),
        ],
    )(psum, psum, y_r, pdeg3, pdeg3,
      b1.reshape(1, h), bn1_w.reshape(1, h), bn1_b.reshape(1, h),
      bn1_m.reshape(1, h), bn1_v.reshape(1, h),
      W2_l, W2_r, Wp, b2.reshape(1, h), bp.reshape(1, 1))

    p2 = _sc_agg_scalar(n_pad, e_pad)(zl.reshape(n), src2d, dst2d)
    p23 = p2.reshape(NC, n_pad, 1)

    out = pl.pallas_call(
        _k3_body,
        grid=(n // bm3,),
        in_specs=[
            pl.BlockSpec((1, bm3, 1), lambda i: (0, i, 0)),
            pl.BlockSpec((1, bm3, 1), lambda i: (1, i, 0)),
            pl.BlockSpec((1, bm3, 1), lambda i: (0, i, 0)),
            pl.BlockSpec((1, bm3, 1), lambda i: (1, i, 0)),
            pl.BlockSpec((bm3, 1), lambda i: (i, 0)),
        ],
        out_specs=pl.BlockSpec((bm3, 1), lambda i: (i, 0)),
        out_shape=jax.ShapeDtypeStruct((n, 1), f32),
    )(p23, p23, pdeg3, pdeg3, zr)
    return out
